# Initial kernel scaffold; baseline (speedup 1.0000x reference)
import jax
import jax.numpy as jnp
from jax.experimental import pallas as pl


def kernel(wild_x, wild_edge_index, wild_batch, mutant_x, mutant_edge_index, mutant_batch, W1w, b1w, W2w, b2w, Wfw, bfw, W1m, b1m, W2m, b2m, Wfm, bfm, Wm1, bm1, Wm2, bm2):
    raise NotImplementedError("write your pallas kernel here")


# speedup vs baseline: 8.5280x; 8.5280x over previous
import functools
import jax
import jax.numpy as jnp
from jax import lax
from jax.experimental import pallas as pl
from jax.experimental.pallas import tpu as pltpu
from jax.experimental.pallas import tpu_sc as plsc

N = 10000
E = 320000
D = 128
OUTD = 2048
INNER = 512
NG = 16

NTILES = 16
CHUNK = 128
NCH = 160
BLK = 16
EPT = NCH * CHUNK
EPAD = NTILES * EPT
NPAD = 10240
RPT = NPAD // NTILES

@functools.lru_cache(maxsize=None)
def _mesh():
    return plsc.VectorSubcoreMesh(core_axis_name="c", subcore_axis_name="s")


@functools.lru_cache(maxsize=None)
def _sc_degree():
    return pl.kernel(
        _sc_degree_body,
        out_type=(
            jax.ShapeDtypeStruct((NPAD, D), jnp.float32),
            jax.ShapeDtypeStruct((NPAD, D), jnp.float32),
        ),
        mesh=_mesh(),
        scratch_types=[
            pltpu.VMEM((NCH, CHUNK), jnp.int32),
            pltpu.VMEM((CHUNK, D), jnp.float32),
            pltpu.VMEM_SHARED((NPAD, D), jnp.float32),
        ],
    )


def _sc_degree_body(dstw, dstm, ones_hbm, zeros_hbm, degw_out, degm_out,
                    dst_v, ones_v, deg_sh):
    c = lax.axis_index("c")
    s = lax.axis_index("s")
    pltpu.sync_copy(ones_hbm, ones_v)
    pltpu.sync_copy(zeros_hbm, deg_sh.at[pl.ds(s * RPT, RPT)])
    plsc.subcore_barrier()

    def run(dst_hbm):
        pltpu.sync_copy(dst_hbm.at[s], dst_v)

        def body(j, carry):
            pltpu.sync_copy(ones_v, deg_sh.at[dst_v.at[j]], add=True)
            return carry

        lax.fori_loop(0, NCH, body, 0)

    def wout(out_hbm):
        def step(t, carry):
            off = s * RPT + t * CHUNK
            pltpu.sync_copy(deg_sh.at[pl.ds(off, CHUNK)], ones_v)
            pltpu.sync_copy(ones_v, out_hbm.at[pl.ds(off, CHUNK)])
            return carry

        lax.fori_loop(0, RPT // CHUNK, step, 0)

    @pl.when(c == 0)
    def _():
        run(dstw)

    @pl.when(c == 1)
    def _():
        run(dstm)

    plsc.subcore_barrier()

    @pl.when(c == 0)
    def _():
        wout(degw_out)

    @pl.when(c == 1)
    def _():
        wout(degm_out)


@functools.lru_cache(maxsize=None)
def _sc_scatter():
    return pl.kernel(
        _sc_scatter_body,
        out_type=(
            jax.ShapeDtypeStruct((NPAD, D), jnp.float32),
            jax.ShapeDtypeStruct((NPAD, D), jnp.float32),
        ),
        mesh=_mesh(),
        scratch_types=[
            pltpu.VMEM((BLK, CHUNK), jnp.int32),
            pltpu.VMEM((BLK, CHUNK), jnp.int32),
            pltpu.VMEM((CHUNK, D), jnp.float32),
            pltpu.VMEM_SHARED((NPAD, D), jnp.float32),
            pltpu.SemaphoreType.DMA,
        ],
    )


def _sc_scatter_body(yw, ym, srcw, dstw, srcm, dstm, zeros_hbm,
                     accw_out, accm_out, src_v, dst_v, rows_v, acc_sh, sem):
    c = lax.axis_index("c")
    s = lax.axis_index("s")
    pltpu.sync_copy(zeros_hbm, acc_sh.at[pl.ds(s * RPT, RPT)])
    plsc.subcore_barrier()

    def run(table_hbm, src_hbm, dst_hbm):
        def outer(b, carry):
            pltpu.sync_copy(src_hbm.at[s, pl.ds(b * BLK, BLK)], src_v)
            pltpu.sync_copy(dst_hbm.at[s, pl.ds(b * BLK, BLK)], dst_v)

            def body(j, carry2):
                pltpu.async_copy(table_hbm.at[src_v.at[j]], rows_v,
                                 sem).wait()
                pltpu.sync_copy(rows_v, acc_sh.at[dst_v.at[j]], add=True)
                return carry2

            lax.fori_loop(0, BLK, body, 0)
            return carry

        lax.fori_loop(0, NCH // BLK, outer, 0)

    def wout(out_hbm):
        def step(t, carry):
            off = s * RPT + t * CHUNK
            pltpu.sync_copy(acc_sh.at[pl.ds(off, CHUNK)], rows_v)
            pltpu.sync_copy(rows_v, out_hbm.at[pl.ds(off, CHUNK)])
            return carry

        lax.fori_loop(0, RPT // CHUNK, step, 0)

    @pl.when(c == 0)
    def _():
        run(yw, srcw, dstw)

    @pl.when(c == 1)
    def _():
        run(ym, srcm, dstm)

    plsc.subcore_barrier()

    @pl.when(c == 0)
    def _():
        wout(accw_out)

    @pl.when(c == 1)
    def _():
        wout(accm_out)


_RB = 1000


def _dinv_body(deg_ref, out_ref):
    v = deg_ref[...]
    d = v[:, :, 0:1] + 1.0
    out_ref[...] = jnp.broadcast_to(lax.rsqrt(d), out_ref.shape)


def _tc_dinv(deg_s):
    return pl.pallas_call(
        _dinv_body,
        grid=(2, N // _RB),
        in_specs=[pl.BlockSpec((1, _RB, D), lambda b, i: (b, i, 0))],
        out_specs=pl.BlockSpec((1, _RB, D), lambda b, i: (b, i, 0)),
        out_shape=jax.ShapeDtypeStruct((2, N, D), jnp.float32),
    )(deg_s)


def _mm1_body(x_ref, w_ref, dinv_ref, out_ref):
    xw = lax.dot_general(x_ref[0], w_ref[0], (((1,), (0,)), ((), ())),
                         preferred_element_type=jnp.float32)
    out_ref[...] = (xw * dinv_ref[0])[None]


def _tc_mm1(x_s, w_s, dinv_b):
    return pl.pallas_call(
        _mm1_body,
        grid=(2, N // _RB),
        in_specs=[
            pl.BlockSpec((1, _RB, D), lambda b, i: (b, i, 0)),
            pl.BlockSpec((1, D, D), lambda b, i: (b, 0, 0)),
            pl.BlockSpec((1, _RB, D), lambda b, i: (b, i, 0)),
        ],
        out_specs=pl.BlockSpec((1, _RB, D), lambda b, i: (b, i, 0)),
        out_shape=jax.ShapeDtypeStruct((2, N, D), jnp.float32),
    )(x_s, w_s, dinv_b)


def _mid_body(acc_ref, y_ref, dinv_ref, b_ref, x_ref, w_ref, out_ref):
    z = (acc_ref[0] + y_ref[0]) * dinv_ref[0] + b_ref[0]
    h = x_ref[0] + jnp.maximum(z, 0.0)
    hw = lax.dot_general(h, w_ref[0], (((1,), (0,)), ((), ())),
                         preferred_element_type=jnp.float32)
    out_ref[...] = (hw * dinv_ref[0])[None]


def _tc_mid(acc_s, y_s, dinv_b, b_s, x_s, w_s):
    return pl.pallas_call(
        _mid_body,
        grid=(2, N // _RB),
        in_specs=[
            pl.BlockSpec((1, _RB, D), lambda b, i: (b, i, 0)),
            pl.BlockSpec((1, _RB, D), lambda b, i: (b, i, 0)),
            pl.BlockSpec((1, _RB, D), lambda b, i: (b, i, 0)),
            pl.BlockSpec((1, 1, D), lambda b, i: (b, 0, 0)),
            pl.BlockSpec((1, _RB, D), lambda b, i: (b, i, 0)),
            pl.BlockSpec((1, D, D), lambda b, i: (b, 0, 0)),
        ],
        out_specs=pl.BlockSpec((1, _RB, D), lambda b, i: (b, i, 0)),
        out_shape=jax.ShapeDtypeStruct((2, N, D), jnp.float32),
    )(acc_s, y_s, dinv_b, b_s, x_s, w_s)


def _pool_body(acc_ref, y_ref, dinv_ref, b_ref, batch_ref, wf_ref, bf_ref,
               out_ref):
    z = (acc_ref[0] + y_ref[0]) * dinv_ref[0] + b_ref[0]
    bat = batch_ref[0]
    neg = jnp.float32(-jnp.inf)
    rows = []
    for g in range(NG):
        zg = jnp.where(bat == g, z, neg)
        rows.append(jnp.max(zg, axis=0))
    pooled = jnp.stack(rows)
    f = lax.dot_general(pooled, wf_ref[0], (((1,), (0,)), ((), ())),
                        preferred_element_type=jnp.float32) + bf_ref[0]
    out_ref[...] = jnp.maximum(f, 0.0)[None]


def _tc_pool(acc_s, y_s, dinv_b, b_s, batch_b, wf_s, bf_s):
    return pl.pallas_call(
        _pool_body,
        grid=(2,),
        in_specs=[
            pl.BlockSpec((1, N, D), lambda b: (b, 0, 0)),
            pl.BlockSpec((1, N, D), lambda b: (b, 0, 0)),
            pl.BlockSpec((1, N, D), lambda b: (b, 0, 0)),
            pl.BlockSpec((1, 1, D), lambda b: (b, 0, 0)),
            pl.BlockSpec((1, N, D), lambda b: (b, 0, 0)),
            pl.BlockSpec((1, D, OUTD), lambda b: (b, 0, 0)),
            pl.BlockSpec((1, 1, OUTD), lambda b: (b, 0, 0)),
        ],
        out_specs=pl.BlockSpec((1, NG, OUTD), lambda b: (b, 0, 0)),
        out_shape=jax.ShapeDtypeStruct((2, NG, OUTD), jnp.float32),
    )(acc_s, y_s, dinv_b, b_s, batch_b, wf_s, bf_s)


def _head_body(xc_ref, w1_ref, b1_ref, w2_ref, b2_ref, out_ref):
    h = lax.dot_general(xc_ref[...], w1_ref[...], (((1,), (0,)), ((), ())),
                        preferred_element_type=jnp.float32) + b1_ref[...]
    h = jnp.maximum(h, 0.0)
    out_ref[...] = lax.dot_general(h, w2_ref[...], (((1,), (0,)), ((), ())),
                                   preferred_element_type=jnp.float32) \
        + b2_ref[...]


def _tc_head(xc, w1, b1, w2p, b2p):
    return pl.pallas_call(
        _head_body,
        out_shape=jax.ShapeDtypeStruct((NG, D), jnp.float32),
    )(xc, w1, b1, w2p, b2p)


def _prep_idx(idx):
    pad = jnp.full((EPAD - E,), N, jnp.int32)
    return jnp.concatenate([idx.astype(jnp.int32), pad]).reshape(
        NTILES, NCH, CHUNK)


def _pad_rows(y):
    return jnp.pad(y, ((0, NPAD - N), (0, 0)))


def kernel(wild_x, wild_edge_index, wild_batch, mutant_x, mutant_edge_index,
           mutant_batch, W1w, b1w, W2w, b2w, Wfw, bfw, W1m, b1m, W2m, b2m,
           Wfm, bfm, Wm1, bm1, Wm2, bm2):
    srcw = _prep_idx(wild_edge_index[0])
    dstw = _prep_idx(wild_edge_index[1])
    srcm = _prep_idx(mutant_edge_index[0])
    dstm = _prep_idx(mutant_edge_index[1])

    ones_in = jnp.ones((CHUNK, D), jnp.float32)
    zeros_in = jnp.zeros((RPT, D), jnp.float32)
    degw, degm = _sc_degree()(dstw, dstm, ones_in, zeros_in)
    deg_s = jnp.stack([degw[:N], degm[:N]])
    dinv_b = _tc_dinv(deg_s)

    x_s = jnp.stack([wild_x, mutant_x])
    w1_s = jnp.stack([W1w, W1m])
    y1 = _tc_mm1(x_s, w1_s, dinv_b)

    acc1w, acc1m = _sc_scatter()(_pad_rows(y1[0]), _pad_rows(y1[1]),
                                 srcw, dstw, srcm, dstm, zeros_in)
    acc1 = jnp.stack([acc1w[:N], acc1m[:N]])

    b1_s = jnp.stack([b1w, b1m])[:, None, :]
    w2_s = jnp.stack([W2w, W2m])
    y2 = _tc_mid(acc1, y1, dinv_b, b1_s, x_s, w2_s)

    acc2w, acc2m = _sc_scatter()(_pad_rows(y2[0]), _pad_rows(y2[1]),
                                 srcw, dstw, srcm, dstm, zeros_in)
    acc2 = jnp.stack([acc2w[:N], acc2m[:N]])

    b2_s = jnp.stack([b2w, b2m])[:, None, :]
    batch_b = jnp.stack([
        jnp.broadcast_to(wild_batch.astype(jnp.int32)[:, None], (N, D)),
        jnp.broadcast_to(mutant_batch.astype(jnp.int32)[:, None], (N, D)),
    ])
    wf_s = jnp.stack([Wfw, Wfm])
    bf_s = jnp.stack([bfw, bfm])[:, None, :]
    f = _tc_pool(acc2, y2, dinv_b, b2_s, batch_b, wf_s, bf_s)

    xc = jnp.concatenate([f[1], f[0]], axis=1)
    w2p = jnp.pad(Wm2, ((0, 0), (0, D - 1)))
    b2p = jnp.pad(bm2, ((0, D - 1)))[None, :]
    out = _tc_head(xc, Wm1, bm1[None, :], w2p, b2p)
    return out[:, :1]


# speedup vs baseline: 9.5691x; 1.1221x over previous
import functools
import jax
import jax.numpy as jnp
from jax import lax
from jax.experimental import pallas as pl
from jax.experimental.pallas import tpu as pltpu
from jax.experimental.pallas import tpu_sc as plsc

N = 10000
E = 320000
D = 128
OUTD = 2048
INNER = 512
NG = 16

NTILES = 16
CHUNK = 128
NCH = 160
BLK = 16
EPT = NCH * CHUNK
EPAD = NTILES * EPT
NPAD = 10240
RPT = NPAD // NTILES

@functools.lru_cache(maxsize=None)
def _mesh():
    return plsc.VectorSubcoreMesh(core_axis_name="c", subcore_axis_name="s")


@functools.lru_cache(maxsize=None)
def _sc_degree():
    return pl.kernel(
        _sc_degree_body,
        out_type=(
            jax.ShapeDtypeStruct((NPAD, D), jnp.float32),
            jax.ShapeDtypeStruct((NPAD, D), jnp.float32),
        ),
        mesh=_mesh(),
        scratch_types=[
            pltpu.VMEM((NCH, CHUNK), jnp.int32),
            pltpu.VMEM((CHUNK, D), jnp.float32),
            pltpu.VMEM_SHARED((NPAD, D), jnp.float32),
        ],
    )


def _sc_degree_body(dstw, dstm, ones_hbm, zeros_hbm, degw_out, degm_out,
                    dst_v, ones_v, deg_sh):
    c = lax.axis_index("c")
    s = lax.axis_index("s")
    pltpu.sync_copy(ones_hbm, ones_v)
    pltpu.sync_copy(zeros_hbm, deg_sh.at[pl.ds(s * RPT, RPT)])
    plsc.subcore_barrier()

    def run(dst_hbm):
        pltpu.sync_copy(dst_hbm.at[s], dst_v)

        def body(j, carry):
            pltpu.sync_copy(ones_v, deg_sh.at[dst_v.at[j]], add=True)
            return carry

        lax.fori_loop(0, NCH, body, 0)

    def wout(out_hbm):
        def step(t, carry):
            off = s * RPT + t * CHUNK
            pltpu.sync_copy(deg_sh.at[pl.ds(off, CHUNK)], ones_v)
            pltpu.sync_copy(ones_v, out_hbm.at[pl.ds(off, CHUNK)])
            return carry

        lax.fori_loop(0, RPT // CHUNK, step, 0)

    @pl.when(c == 0)
    def _():
        run(dstw)

    @pl.when(c == 1)
    def _():
        run(dstm)

    plsc.subcore_barrier()

    @pl.when(c == 0)
    def _():
        wout(degw_out)

    @pl.when(c == 1)
    def _():
        wout(degm_out)


@functools.lru_cache(maxsize=None)
def _sc_scatter():
    return pl.kernel(
        _sc_scatter_body,
        out_type=(
            jax.ShapeDtypeStruct((NPAD, D), jnp.float32),
            jax.ShapeDtypeStruct((NPAD, D), jnp.float32),
        ),
        mesh=_mesh(),
        scratch_types=[
            pltpu.VMEM((2, BLK, CHUNK), jnp.int32),
            pltpu.VMEM((2, BLK, CHUNK), jnp.int32),
            pltpu.VMEM((2, CHUNK, D), jnp.float32),
            pltpu.VMEM_SHARED((NPAD, D), jnp.float32),
            pltpu.SemaphoreType.DMA,
            pltpu.SemaphoreType.DMA,
            pltpu.SemaphoreType.DMA,
        ],
    )


def _sc_scatter_body(yw, ym, srcw, dstw, srcm, dstm, zeros_hbm,
                     accw_out, accm_out, src_v, dst_v, rows_v, acc_sh,
                     gsem0, gsem1, isem):
    c = lax.axis_index("c")
    s = lax.axis_index("s")
    NBLK = NCH // BLK
    gsem = (gsem0, gsem1)
    pltpu.sync_copy(zeros_hbm, acc_sh.at[pl.ds(s * RPT, RPT)])
    plsc.subcore_barrier()

    def run(table_hbm, src_hbm, dst_hbm):
        pltpu.sync_copy(src_hbm.at[s, pl.ds(0, BLK)], src_v.at[0])
        pltpu.sync_copy(dst_hbm.at[s, pl.ds(0, BLK)], dst_v.at[0])
        pltpu.async_copy(table_hbm.at[src_v.at[0].at[0]], rows_v.at[0],
                         gsem[0])

        def block_pair(i, carry):
            for bb in range(2):
                b = 2 * i + bb
                nbb = (bb + 1) % 2

                @pl.when(b + 1 < NBLK)
                def _():
                    pltpu.async_copy(
                        src_hbm.at[s, pl.ds((b + 1) * BLK, BLK)],
                        src_v.at[nbb], isem)
                    pltpu.async_copy(
                        dst_hbm.at[s, pl.ds((b + 1) * BLK, BLK)],
                        dst_v.at[nbb], isem)

                for j in range(BLK):
                    pj = j % 2
                    nj = (j + 1) % 2
                    pltpu.make_async_copy(
                        table_hbm.at[src_v.at[bb].at[j]], rows_v.at[pj],
                        gsem[pj]).wait()
                    if j + 1 < BLK:
                        pltpu.async_copy(
                            table_hbm.at[src_v.at[bb].at[j + 1]],
                            rows_v.at[nj], gsem[nj])
                    else:
                        @pl.when(b + 1 < NBLK)
                        def _():
                            pltpu.make_async_copy(
                                src_hbm.at[s, pl.ds((b + 1) * BLK, BLK)],
                                src_v.at[nbb], isem).wait()
                            pltpu.make_async_copy(
                                dst_hbm.at[s, pl.ds((b + 1) * BLK, BLK)],
                                dst_v.at[nbb], isem).wait()
                            pltpu.async_copy(
                                table_hbm.at[src_v.at[nbb].at[0]],
                                rows_v.at[nj], gsem[nj])
                    pltpu.sync_copy(rows_v.at[pj],
                                    acc_sh.at[dst_v.at[bb].at[j]],
                                    add=True)
            return carry

        lax.fori_loop(0, NBLK // 2, block_pair, 0)

    def wout(out_hbm):
        def step(t, carry):
            off = s * RPT + t * CHUNK
            pltpu.sync_copy(acc_sh.at[pl.ds(off, CHUNK)], rows_v.at[0])
            pltpu.sync_copy(rows_v.at[0], out_hbm.at[pl.ds(off, CHUNK)])
            return carry

        lax.fori_loop(0, RPT // CHUNK, step, 0)

    @pl.when(c == 0)
    def _():
        run(yw, srcw, dstw)

    @pl.when(c == 1)
    def _():
        run(ym, srcm, dstm)

    plsc.subcore_barrier()

    @pl.when(c == 0)
    def _():
        wout(accw_out)

    @pl.when(c == 1)
    def _():
        wout(accm_out)


_RB = 1000


def _dinv_body(deg_ref, out_ref):
    v = deg_ref[...]
    d = v[:, :, 0:1] + 1.0
    out_ref[...] = jnp.broadcast_to(lax.rsqrt(d), out_ref.shape)


def _tc_dinv(deg_s):
    return pl.pallas_call(
        _dinv_body,
        grid=(2, N // _RB),
        in_specs=[pl.BlockSpec((1, _RB, D), lambda b, i: (b, i, 0))],
        out_specs=pl.BlockSpec((1, _RB, D), lambda b, i: (b, i, 0)),
        out_shape=jax.ShapeDtypeStruct((2, N, D), jnp.float32),
    )(deg_s)


def _mm1_body(x_ref, w_ref, dinv_ref, out_ref):
    xw = lax.dot_general(x_ref[0], w_ref[0], (((1,), (0,)), ((), ())),
                         preferred_element_type=jnp.float32)
    out_ref[...] = (xw * dinv_ref[0])[None]


def _tc_mm1(x_s, w_s, dinv_b):
    return pl.pallas_call(
        _mm1_body,
        grid=(2, N // _RB),
        in_specs=[
            pl.BlockSpec((1, _RB, D), lambda b, i: (b, i, 0)),
            pl.BlockSpec((1, D, D), lambda b, i: (b, 0, 0)),
            pl.BlockSpec((1, _RB, D), lambda b, i: (b, i, 0)),
        ],
        out_specs=pl.BlockSpec((1, _RB, D), lambda b, i: (b, i, 0)),
        out_shape=jax.ShapeDtypeStruct((2, N, D), jnp.float32),
    )(x_s, w_s, dinv_b)


def _mid_body(acc_ref, y_ref, dinv_ref, b_ref, x_ref, w_ref, out_ref):
    z = (acc_ref[0] + y_ref[0]) * dinv_ref[0] + b_ref[0]
    h = x_ref[0] + jnp.maximum(z, 0.0)
    hw = lax.dot_general(h, w_ref[0], (((1,), (0,)), ((), ())),
                         preferred_element_type=jnp.float32)
    out_ref[...] = (hw * dinv_ref[0])[None]


def _tc_mid(acc_s, y_s, dinv_b, b_s, x_s, w_s):
    return pl.pallas_call(
        _mid_body,
        grid=(2, N // _RB),
        in_specs=[
            pl.BlockSpec((1, _RB, D), lambda b, i: (b, i, 0)),
            pl.BlockSpec((1, _RB, D), lambda b, i: (b, i, 0)),
            pl.BlockSpec((1, _RB, D), lambda b, i: (b, i, 0)),
            pl.BlockSpec((1, 1, D), lambda b, i: (b, 0, 0)),
            pl.BlockSpec((1, _RB, D), lambda b, i: (b, i, 0)),
            pl.BlockSpec((1, D, D), lambda b, i: (b, 0, 0)),
        ],
        out_specs=pl.BlockSpec((1, _RB, D), lambda b, i: (b, i, 0)),
        out_shape=jax.ShapeDtypeStruct((2, N, D), jnp.float32),
    )(acc_s, y_s, dinv_b, b_s, x_s, w_s)


def _pool_body(acc_ref, y_ref, dinv_ref, b_ref, batch_ref, wf_ref, bf_ref,
               out_ref):
    z = (acc_ref[0] + y_ref[0]) * dinv_ref[0] + b_ref[0]
    bat = batch_ref[0]
    neg = jnp.float32(-jnp.inf)
    rows = []
    for g in range(NG):
        zg = jnp.where(bat == g, z, neg)
        rows.append(jnp.max(zg, axis=0))
    pooled = jnp.stack(rows)
    f = lax.dot_general(pooled, wf_ref[0], (((1,), (0,)), ((), ())),
                        preferred_element_type=jnp.float32) + bf_ref[0]
    out_ref[...] = jnp.maximum(f, 0.0)[None]


def _tc_pool(acc_s, y_s, dinv_b, b_s, batch_b, wf_s, bf_s):
    return pl.pallas_call(
        _pool_body,
        grid=(2,),
        in_specs=[
            pl.BlockSpec((1, N, D), lambda b: (b, 0, 0)),
            pl.BlockSpec((1, N, D), lambda b: (b, 0, 0)),
            pl.BlockSpec((1, N, D), lambda b: (b, 0, 0)),
            pl.BlockSpec((1, 1, D), lambda b: (b, 0, 0)),
            pl.BlockSpec((1, N, D), lambda b: (b, 0, 0)),
            pl.BlockSpec((1, D, OUTD), lambda b: (b, 0, 0)),
            pl.BlockSpec((1, 1, OUTD), lambda b: (b, 0, 0)),
        ],
        out_specs=pl.BlockSpec((1, NG, OUTD), lambda b: (b, 0, 0)),
        out_shape=jax.ShapeDtypeStruct((2, NG, OUTD), jnp.float32),
    )(acc_s, y_s, dinv_b, b_s, batch_b, wf_s, bf_s)


def _head_body(xc_ref, w1_ref, b1_ref, w2_ref, b2_ref, out_ref):
    h = lax.dot_general(xc_ref[...], w1_ref[...], (((1,), (0,)), ((), ())),
                        preferred_element_type=jnp.float32) + b1_ref[...]
    h = jnp.maximum(h, 0.0)
    out_ref[...] = lax.dot_general(h, w2_ref[...], (((1,), (0,)), ((), ())),
                                   preferred_element_type=jnp.float32) \
        + b2_ref[...]


def _tc_head(xc, w1, b1, w2p, b2p):
    return pl.pallas_call(
        _head_body,
        out_shape=jax.ShapeDtypeStruct((NG, D), jnp.float32),
    )(xc, w1, b1, w2p, b2p)


def _prep_idx(idx):
    pad = jnp.full((EPAD - E,), N, jnp.int32)
    return jnp.concatenate([idx.astype(jnp.int32), pad]).reshape(
        NTILES, NCH, CHUNK)


def _pad_rows(y):
    return jnp.pad(y, ((0, NPAD - N), (0, 0)))


def kernel(wild_x, wild_edge_index, wild_batch, mutant_x, mutant_edge_index,
           mutant_batch, W1w, b1w, W2w, b2w, Wfw, bfw, W1m, b1m, W2m, b2m,
           Wfm, bfm, Wm1, bm1, Wm2, bm2):
    srcw = _prep_idx(wild_edge_index[0])
    dstw = _prep_idx(wild_edge_index[1])
    srcm = _prep_idx(mutant_edge_index[0])
    dstm = _prep_idx(mutant_edge_index[1])

    ones_in = jnp.ones((CHUNK, D), jnp.float32)
    zeros_in = jnp.zeros((RPT, D), jnp.float32)
    degw, degm = _sc_degree()(dstw, dstm, ones_in, zeros_in)
    deg_s = jnp.stack([degw[:N], degm[:N]])
    dinv_b = _tc_dinv(deg_s)

    x_s = jnp.stack([wild_x, mutant_x])
    w1_s = jnp.stack([W1w, W1m])
    y1 = _tc_mm1(x_s, w1_s, dinv_b)

    acc1w, acc1m = _sc_scatter()(_pad_rows(y1[0]), _pad_rows(y1[1]),
                                 srcw, dstw, srcm, dstm, zeros_in)
    acc1 = jnp.stack([acc1w[:N], acc1m[:N]])

    b1_s = jnp.stack([b1w, b1m])[:, None, :]
    w2_s = jnp.stack([W2w, W2m])
    y2 = _tc_mid(acc1, y1, dinv_b, b1_s, x_s, w2_s)

    acc2w, acc2m = _sc_scatter()(_pad_rows(y2[0]), _pad_rows(y2[1]),
                                 srcw, dstw, srcm, dstm, zeros_in)
    acc2 = jnp.stack([acc2w[:N], acc2m[:N]])

    b2_s = jnp.stack([b2w, b2m])[:, None, :]
    batch_b = jnp.stack([
        jnp.broadcast_to(wild_batch.astype(jnp.int32)[:, None], (N, D)),
        jnp.broadcast_to(mutant_batch.astype(jnp.int32)[:, None], (N, D)),
    ])
    wf_s = jnp.stack([Wfw, Wfm])
    bf_s = jnp.stack([bfw, bfm])[:, None, :]
    f = _tc_pool(acc2, y2, dinv_b, b2_s, batch_b, wf_s, bf_s)

    xc = jnp.concatenate([f[1], f[0]], axis=1)
    w2p = jnp.pad(Wm2, ((0, 0), (0, D - 1)))
    b2p = jnp.pad(bm2, ((0, D - 1)))[None, :]
    out = _tc_head(xc, Wm1, bm1[None, :], w2p, b2p)
    return out[:, :1]
